# baseline (speedup 1.0000x reference)
import functools

import jax
import jax.numpy as jnp
from jax import lax
from jax.experimental import pallas as pl
from jax.experimental.pallas import tpu as pltpu
from jax.experimental.pallas import tpu_sc as plsc

N = 10000
E = 320000
D = 128

NC = 2
NS = 16
NW = NC * NS
CHUNK = 128
ITERS = 80
G = 8
GROUPS = ITERS // G
NCHUNKS = NW * ITERS
EPAD = NCHUNKS * CHUNK
NPAD = 10240
ROWS_PER_SUB = NPAD // NS
SWEEP = ROWS_PER_SUB // CHUNK
DUMMY_ROW = NPAD - 1

_MESH = dict(core_axis_name="c", subcore_axis_name="s",
             num_cores=NC, num_subcores=NS)


def _ids():
  cid = lax.axis_index("c")
  sid = lax.axis_index("s")
  return cid, sid, sid * NC + cid, sid * ROWS_PER_SUB


def _zero_acc(zrows_hbm, rows_v, acc_sh, row0):
  pltpu.sync_copy(zrows_hbm, rows_v)
  for k in range(SWEEP):
    pltpu.sync_copy(rows_v, acc_sh.at[pl.ds(row0 + k * CHUNK, CHUNK)])


def _write_back(acc_sh, rows_v, out_hbm, cid, row0):
  for k in range(SWEEP):
    r = row0 + k * CHUNK
    pltpu.sync_copy(acc_sh.at[pl.ds(r, CHUNK)], rows_v)
    pltpu.sync_copy(rows_v, out_hbm.at[cid, pl.ds(r, CHUNK)])


@functools.lru_cache(maxsize=None)
def _make_sc_agg():
  out_type = [jax.ShapeDtypeStruct((NC, NPAD, D), jnp.float32)]
  scratch = [
      pltpu.VMEM((G, CHUNK), jnp.int32),
      pltpu.VMEM((G, CHUNK), jnp.int32),
      pltpu.VMEM((CHUNK, D), jnp.float32),
      pltpu.VMEM((CHUNK, D), jnp.float32),
      pltpu.SemaphoreType.DMA,
      pltpu.SemaphoreType.DMA,
      pltpu.SemaphoreType.DMA,
      pltpu.SemaphoreType.DMA,
      pltpu.SemaphoreType.DMA,
      pltpu.SemaphoreType.DMA,
      pltpu.VMEM_SHARED((NPAD, D), jnp.float32),
  ]

  def body(h_hbm, src_hbm, dst_hbm, zrows_hbm, agg_hbm,
           sblk, dblk, rows0, rows1, si, di, gt0, gt1, st0, st1, acc_sh):
    cid, sid, wid, row0 = _ids()
    _zero_acc(zrows_hbm, rows0, acc_sh, row0)
    plsc.subcore_barrier()

    rows = (rows0, rows1)
    gt = (gt0, gt1)
    st = (st0, st1)
    base = wid * ITERS

    def idx_fetch(i):
      off = base + i * G
      pltpu.async_copy(src_hbm.at[pl.ds(off, G)], sblk, si)
      pltpu.async_copy(dst_hbm.at[pl.ds(off, G)], dblk, di)

    def idx_wait(i):
      off = base + i * G
      pltpu.make_async_copy(src_hbm.at[pl.ds(off, G)], sblk, si).wait()
      pltpu.make_async_copy(dst_hbm.at[pl.ds(off, G)], dblk, di).wait()

    def gather(g, b):
      pltpu.async_copy(h_hbm.at[sblk.at[g]], rows[b], gt[b])

    def gather_wait(g, b):
      pltpu.make_async_copy(h_hbm.at[sblk.at[g]], rows[b], gt[b]).wait()

    def scatter(g, b):
      pltpu.async_copy(rows[b], acc_sh.at[dblk.at[g]], st[b], add=True)

    def scatter_wait(g, b):
      pltpu.make_async_copy(rows[b], acc_sh.at[dblk.at[g]], st[b]).wait()

    def do_group(i, fetch_next):
      idx_wait(i)
      gather(0, 0)
      gather(1, 1)
      for g in range(G):
        b = g % 2
        gather_wait(g, b)
        scatter(g, b)
        if g + 2 < G:
          scatter_wait(g, b)
          gather(g + 2, b)
      scatter_wait(G - 2, 0)
      scatter_wait(G - 1, 1)
      if fetch_next:
        idx_fetch(i + 1)

    idx_fetch(0)

    def grp_step(i, carry):
      do_group(i, True)
      return carry

    lax.fori_loop(0, GROUPS - 1, grp_step, 0)
    do_group(GROUPS - 1, False)

    plsc.subcore_barrier()
    _write_back(acc_sh, rows0, agg_hbm, cid, row0)

  return pl.kernel(body, out_type=out_type,
                   mesh=plsc.VectorSubcoreMesh(**_MESH),
                   scratch_types=scratch)


@functools.lru_cache(maxsize=None)
def _make_sc_deg():
  out_type = [jax.ShapeDtypeStruct((NC, NPAD, D), jnp.float32)]
  scratch = [
      pltpu.VMEM((G, CHUNK), jnp.int32),
      pltpu.VMEM((CHUNK, D), jnp.float32),
      pltpu.SemaphoreType.DMA,
      pltpu.SemaphoreType.DMA,
      pltpu.VMEM_SHARED((NPAD, D), jnp.float32),
  ]

  def body(dst_hbm, zrows_hbm, ones_hbm, deg_hbm,
           dblk, rows_v, di, st, acc_sh):
    cid, sid, wid, row0 = _ids()
    _zero_acc(zrows_hbm, rows_v, acc_sh, row0)
    plsc.subcore_barrier()
    pltpu.sync_copy(ones_hbm, rows_v)
    base = wid * ITERS

    def idx_fetch(i):
      off = base + i * G
      pltpu.async_copy(dst_hbm.at[pl.ds(off, G)], dblk, di)

    def idx_wait(i):
      off = base + i * G
      pltpu.make_async_copy(dst_hbm.at[pl.ds(off, G)], dblk, di).wait()

    def do_group(i, fetch_next):
      idx_wait(i)
      for g in range(G):
        pltpu.async_copy(rows_v, acc_sh.at[dblk.at[g]], st, add=True)
      for g in range(G):
        pltpu.make_async_copy(rows_v, acc_sh.at[dblk.at[g]], st).wait()
      if fetch_next:
        idx_fetch(i + 1)

    idx_fetch(0)

    def grp_step(i, carry):
      do_group(i, True)
      return carry

    lax.fori_loop(0, GROUPS - 1, grp_step, 0)
    do_group(GROUPS - 1, False)

    plsc.subcore_barrier()
    _write_back(acc_sh, rows_v, deg_hbm, cid, row0)

  return pl.kernel(body, out_type=out_type,
                   mesh=plsc.VectorSubcoreMesh(**_MESH),
                   scratch_types=scratch)


_TC_R = 512


def _tc_layer_body(h_ref, p0_ref, p1_ref, d0_ref, d1_ref, w_ref,
                   b_ref, o_ref):
  inv = 1.0 / (d0_ref[:, 0:1] + d1_ref[:, 0:1] + 1.0)
  s = (h_ref[...] + p0_ref[...] + p1_ref[...]) * inv
  o = jnp.dot(s, w_ref[...], preferred_element_type=jnp.float32) + b_ref[...]
  o_ref[...] = jnp.maximum(o, 0.0)


def _tc_layer(h, p0, p1, d0, d1, W, b2d):
  return pl.pallas_call(
      _tc_layer_body,
      grid=(NPAD // _TC_R,),
      in_specs=[
          pl.BlockSpec((_TC_R, D), lambda i: (i, 0)),
          pl.BlockSpec((_TC_R, D), lambda i: (i, 0)),
          pl.BlockSpec((_TC_R, D), lambda i: (i, 0)),
          pl.BlockSpec((_TC_R, D), lambda i: (i, 0)),
          pl.BlockSpec((_TC_R, D), lambda i: (i, 0)),
          pl.BlockSpec((D, D), lambda i: (0, 0)),
          pl.BlockSpec((1, D), lambda i: (0, 0)),
      ],
      out_specs=pl.BlockSpec((_TC_R, D), lambda i: (i, 0)),
      out_shape=jax.ShapeDtypeStruct((N, D), jnp.float32),
  )(h, p0, p1, d0, d1, W, b2d)


def kernel(x, edge_index, W1, b1, W2, b2):
  npad = EPAD - E
  src = jnp.concatenate(
      [edge_index[0].astype(jnp.int32), jnp.zeros((npad,), jnp.int32)])
  dst = jnp.concatenate(
      [edge_index[1].astype(jnp.int32),
       jnp.full((npad,), DUMMY_ROW, jnp.int32)])
  src = src.reshape(NCHUNKS, CHUNK)
  dst = dst.reshape(NCHUNKS, CHUNK)
  zrows = jnp.zeros((CHUNK, D), jnp.float32)
  ones = jnp.ones((CHUNK, D), jnp.float32)

  (deg,) = _make_sc_deg()(dst, zrows, ones)
  sc_agg = _make_sc_agg()
  (agg1,) = sc_agg(x, src, dst, zrows)
  h1 = _tc_layer(x, agg1[0], agg1[1], deg[0], deg[1], W1, b1.reshape(1, D))
  (agg2,) = sc_agg(h1, src, dst, zrows)
  h2 = _tc_layer(h1, agg2[0], agg2[1], deg[0], deg[1], W2, b2.reshape(1, D))
  return h2
